# baseline (speedup 1.0000x reference)
import jax
import jax.numpy as jnp
import numpy as np
from jax.experimental import pallas as pl
from jax.experimental.pallas import tpu as pltpu

_B, _T, _DIMS, _NHEADS = 1, 2048, 768, 12
_HD = _DIMS // _NHEADS
_TAU = 0.1

_RP = 256
_R = 256
_PRE = 8
_M = _R + _PRE
_CH = 8
_NC = _M // _CH


def _proj_kernel(x_ref, w_ref, q_ref, k_ref, v_ref):
    x = x_ref[...]
    y = _dot3(x, w_ref[...])
    for t, o_ref in enumerate((q_ref, k_ref, v_ref)):
        for h in range(_NHEADS):
            sl = y[:, (t * _NHEADS + h) * _HD:(t * _NHEADS + h + 1) * _HD]
            sl = sl * (1.0 / _TAU)
            sl = sl - jnp.max(sl, axis=1, keepdims=True)
            e = jnp.exp(sl)
            o_ref[h] = e / jnp.sum(e, axis=1, keepdims=True)


def _shift_diag(x, s, fill):
    m, t = x.shape
    x = jnp.concatenate([jnp.full((s, t), fill, x.dtype), x[:m - s, :]], axis=0)
    x = jnp.concatenate([jnp.full((m, s), fill, x.dtype), x[:, :t - s]], axis=1)
    return x


def _shift_cols(x, s, fill):
    sh = x.shape[:-1] + (s,)
    return jnp.concatenate(
        [jnp.full(sh, fill, x.dtype), x[..., :x.shape[-1] - s]], axis=-1)


def _dot3(a, bt):
    ah = a.astype(jnp.bfloat16)
    al = (a - ah.astype(jnp.float32)).astype(jnp.bfloat16)
    bh = bt.astype(jnp.bfloat16)
    bl = (bt - bh.astype(jnp.float32)).astype(jnp.bfloat16)
    dims = (((1,), (1,)), ((), ()))
    hi = jax.lax.dot_general(ah, bh, dims, preferred_element_type=jnp.float32)
    m1 = jax.lax.dot_general(ah, bl, dims, preferred_element_type=jnp.float32)
    m2 = jax.lax.dot_general(al, bh, dims, preferred_element_type=jnp.float32)
    return hi + (m1 + m2)


_NG = (_NC + _CH - 1) // _CH


def _scan_stripe(g, b, hb, width):
    g3 = g.reshape(hb * _NC, _CH, width)
    b3 = b.reshape(hb * _NC, _CH, width)
    rnp = jax.lax.broadcasted_iota(jnp.int32, (1, _CH, width), 1)
    cnp = jax.lax.broadcasted_iota(jnp.int32, (1, _CH, width), 2)

    for s in (1, 2, 4):
        fill = (rnp < s) | (cnp < s)
        gs = jnp.where(fill, 1.0, pltpu.roll(pltpu.roll(g3, s, 1), s, 2))
        bs = jnp.where(fill, 0.0, pltpu.roll(pltpu.roll(b3, s, 1), s, 2))
        b3 = g3 * bs + b3
        g3 = g3 * gs

    pad = _NG * _CH - _NC
    lastrow = g3.reshape(hb, _NC, _CH, width)[:, :, _CH - 1, :]
    gc = jnp.concatenate(
        [lastrow, jnp.ones((hb, pad, width), jnp.float32)],
        axis=1).reshape(hb, _NG, _CH, width)
    lastrow = b3.reshape(hb, _NC, _CH, width)[:, :, _CH - 1, :]
    bc = jnp.concatenate(
        [lastrow, jnp.zeros((hb, pad, width), jnp.float32)],
        axis=1).reshape(hb, _NG, _CH, width)
    rnp4 = rnp[None]
    cnp4 = cnp[None]

    def flat_rowshift(x, s, fill):
        gsh, rsh = s // _CH, s % _CH

        def gshift(y, n):
            if n == 0:
                return y
            return jnp.concatenate(
                [jnp.full((hb, n, _CH, width), fill, y.dtype),
                 y[:, :_NG - n]], axis=1)

        if rsh == 0:
            return gshift(x, gsh)
        xr = pltpu.roll(x, rsh, 2)
        return jnp.where(rnp4 < rsh, gshift(xr, gsh + 1), gshift(xr, gsh))

    s = 1
    while s < _NC:
        cs = _CH * s
        gcs = flat_rowshift(gc, s, 1.0)
        bcs = flat_rowshift(bc, s, 0.0)
        cfill = cnp4 < cs
        gcs = jnp.where(cfill, 1.0, pltpu.roll(gcs, cs, 3))
        bcs = jnp.where(cfill, 0.0, pltpu.roll(bcs, cs, 3))
        bc = gc * bcs + bc
        gc = gc * gcs
        s *= 2

    pb = bc.reshape(hb, _NG * _CH, width)
    eb = jnp.concatenate(
        [jnp.zeros((hb, 1, width), jnp.float32), pb[:, :_NC - 1, :]], axis=1)
    w = jnp.broadcast_to(eb[:, :, None, :], (hb, _NC, _CH, width))
    w = w.reshape(hb * _NC, _CH, width)
    w = pltpu.roll(w, 1, 2, stride=1, stride_axis=1)
    w = jnp.where(cnp <= rnp, 0.0, w)
    b3 = g3 * w + b3
    return b3.reshape(hb, _M, width)


def _make_attn_kernel(w, rb0, hb):
    def attn(q_ref, k_ref, v_ref, wo_ref, ci_ref, o_ref, co_ref):
        a = [_dot3(q_ref[i], k_ref[i]) for i in range(hb)]
        a = jnp.concatenate([x[None] for x in a], axis=0)

        carry = ci_ref[:, 0:1, :w]
        carry = jnp.concatenate(
            [carry[:, :, _PRE - 1:], jnp.zeros((hb, 1, _PRE - 1), jnp.float32)],
            axis=2)

        g = jnp.concatenate(
            [jnp.zeros((hb, 1, w), jnp.float32),
             jnp.ones((hb, _PRE - 1, w), jnp.float32), a], axis=1)
        b = jnp.concatenate(
            [carry, jnp.zeros((hb, _PRE - 1, w), jnp.float32), a], axis=1)

        b = _scan_stripe(g, b, hb, w)

        last = b[:, _M - 1:_M, :]
        if w == _T:
            co_ref[...] = jnp.broadcast_to(last, (hb, _CH, _T))
        else:
            co_ref[...] = jnp.concatenate(
                [jnp.broadcast_to(last, (hb, _CH, w)),
                 jnp.zeros((hb, _CH, _T - w), jnp.float32)], axis=2)
        y = b[:, _PRE:, :]

        rows = rb0 * _R + jax.lax.broadcasted_iota(jnp.int32, (1, _R, w), 1)
        cols = jax.lax.broadcasted_iota(jnp.int32, (1, _R, w), 2)
        colsf = cols.astype(jnp.float32)
        rinv = (1.0 / _TAU) / (
            rb0 * _R + 1.0
            + jax.lax.broadcasted_iota(jnp.int32, (1, _R, 1), 1)
            .astype(jnp.float32))
        z = jnp.where(cols <= rows, y * (1.0 / _TAU) + colsf * rinv, -jnp.inf)
        z = z - jnp.max(z, axis=2, keepdims=True)
        e = jnp.exp(z)
        probs = e / jnp.sum(e, axis=2, keepdims=True)

        res = [jnp.dot(jnp.dot(probs[i], v_ref[i],
                               preferred_element_type=jnp.float32),
                       wo_ref[i], preferred_element_type=jnp.float32)
               for i in range(hb)]
        o_ref[...] = jnp.concatenate(res, axis=1)

    return attn


def kernel(x, wq, wk, wv, wo):
    x2 = x.reshape(_T, _DIMS)

    wcat = jnp.concatenate([wq, wk, wv], axis=0)
    q_sm, k_sm, v_sm = pl.pallas_call(
        _proj_kernel,
        grid=(_T // _RP,),
        in_specs=[
            pl.BlockSpec((_RP, _DIMS), lambda rb: (rb, 0)),
            pl.BlockSpec((3 * _DIMS, _DIMS), lambda rb: (0, 0)),
        ],
        out_specs=[
            pl.BlockSpec((_NHEADS, _RP, _HD), lambda rb: (0, rb, 0)),
            pl.BlockSpec((_NHEADS, _RP, _HD), lambda rb: (0, rb, 0)),
            pl.BlockSpec((_NHEADS, _RP, _HD), lambda rb: (0, rb, 0)),
        ],
        out_shape=[jax.ShapeDtypeStruct((_NHEADS, _T, _HD), jnp.float32)] * 3,
    )(x2, wcat)

    wo3 = wo.reshape(_NHEADS, _HD, _HD)
    ci = jnp.zeros((_NHEADS, _CH, _T), jnp.float32)
    parts = []
    for rb0 in range(_T // _R):
        w = _R * (rb0 + 1)
        hb = 6 if w <= 256 else (4 if w <= 1024 else 2)
        out_p, co = pl.pallas_call(
            _make_attn_kernel(w, rb0, hb),
            grid=(_NHEADS // hb,),
            in_specs=[
                pl.BlockSpec((hb, _R, _HD), lambda i, rb0=rb0: (i, rb0, 0)),
                pl.BlockSpec((hb, w, _HD), lambda i: (i, 0, 0)),
                pl.BlockSpec((hb, w, _HD), lambda i: (i, 0, 0)),
                pl.BlockSpec((hb, _HD, _HD), lambda i: (i, 0, 0)),
                pl.BlockSpec((hb, _CH, _T), lambda i: (i, 0, 0)),
            ],
            out_specs=[
                pl.BlockSpec((_R, hb * _HD), lambda i: (0, i)),
                pl.BlockSpec((hb, _CH, _T), lambda i: (i, 0, 0)),
            ],
            out_shape=[
                jax.ShapeDtypeStruct((_R, _DIMS), jnp.float32),
                jax.ShapeDtypeStruct((_NHEADS, _CH, _T), jnp.float32),
            ],
        )(q_sm, k_sm, v_sm, wo3, ci)
        parts.append(out_p)
        ci = co

    return jnp.concatenate(parts, axis=0).reshape(_B, _T, _DIMS)


# speedup vs baseline: 1.0958x; 1.0958x over previous
import jax
import jax.numpy as jnp
import numpy as np
from jax.experimental import pallas as pl
from jax.experimental.pallas import tpu as pltpu

_B, _T, _DIMS, _NHEADS = 1, 2048, 768, 12
_HD = _DIMS // _NHEADS
_TAU = 0.1

_RP = 256
_R = 256
_PRE = 8
_M = _R + _PRE
_CH = 8
_NC = _M // _CH


def _proj_kernel(x_ref, w_ref, q_ref, k_ref, v_ref):
    x = x_ref[...]
    y = jax.lax.dot_general(x, w_ref[...], (((1,), (1,)), ((), ())),
                            preferred_element_type=jnp.float32)
    for t, o_ref in enumerate((q_ref, k_ref, v_ref)):
        for h in range(_NHEADS):
            sl = y[:, (t * _NHEADS + h) * _HD:(t * _NHEADS + h + 1) * _HD]
            sl = sl * (1.0 / _TAU)
            sl = sl - jnp.max(sl, axis=1, keepdims=True)
            e = jnp.exp(sl)
            o_ref[h] = e / jnp.sum(e, axis=1, keepdims=True)


def _shift_diag(x, s, fill):
    m, t = x.shape
    x = jnp.concatenate([jnp.full((s, t), fill, x.dtype), x[:m - s, :]], axis=0)
    x = jnp.concatenate([jnp.full((m, s), fill, x.dtype), x[:, :t - s]], axis=1)
    return x


def _shift_cols(x, s, fill):
    sh = x.shape[:-1] + (s,)
    return jnp.concatenate(
        [jnp.full(sh, fill, x.dtype), x[..., :x.shape[-1] - s]], axis=-1)


_NG = (_NC + _CH - 1) // _CH


def _scan_stripe(g, b, hb, width):
    g3 = g.reshape(hb * _NC, _CH, width)
    b3 = b.reshape(hb * _NC, _CH, width)
    rnp = jax.lax.broadcasted_iota(jnp.int32, (1, _CH, width), 1)
    cnp = jax.lax.broadcasted_iota(jnp.int32, (1, _CH, width), 2)

    for s in (1, 2, 4):
        fill = (rnp < s) | (cnp < s)
        gs = jnp.where(fill, 1.0, pltpu.roll(pltpu.roll(g3, s, 1), s, 2))
        bs = jnp.where(fill, 0.0, pltpu.roll(pltpu.roll(b3, s, 1), s, 2))
        b3 = g3 * bs + b3
        g3 = g3 * gs

    pad = _NG * _CH - _NC
    lastrow = g3.reshape(hb, _NC, _CH, width)[:, :, _CH - 1, :]
    gc = jnp.concatenate(
        [lastrow, jnp.ones((hb, pad, width), jnp.float32)],
        axis=1).reshape(hb, _NG, _CH, width)
    lastrow = b3.reshape(hb, _NC, _CH, width)[:, :, _CH - 1, :]
    bc = jnp.concatenate(
        [lastrow, jnp.zeros((hb, pad, width), jnp.float32)],
        axis=1).reshape(hb, _NG, _CH, width)
    rnp4 = rnp[None]
    cnp4 = cnp[None]

    def flat_rowshift(x, s, fill):
        gsh, rsh = s // _CH, s % _CH

        def gshift(y, n):
            if n == 0:
                return y
            return jnp.concatenate(
                [jnp.full((hb, n, _CH, width), fill, y.dtype),
                 y[:, :_NG - n]], axis=1)

        if rsh == 0:
            return gshift(x, gsh)
        xr = pltpu.roll(x, rsh, 2)
        return jnp.where(rnp4 < rsh, gshift(xr, gsh + 1), gshift(xr, gsh))

    s = 1
    while s < _NC:
        cs = _CH * s
        gcs = flat_rowshift(gc, s, 1.0)
        bcs = flat_rowshift(bc, s, 0.0)
        cfill = cnp4 < cs
        gcs = jnp.where(cfill, 1.0, pltpu.roll(gcs, cs, 3))
        bcs = jnp.where(cfill, 0.0, pltpu.roll(bcs, cs, 3))
        bc = gc * bcs + bc
        gc = gc * gcs
        s *= 2

    pb = bc.reshape(hb, _NG * _CH, width)
    eb = jnp.concatenate(
        [jnp.zeros((hb, 1, width), jnp.float32), pb[:, :_NC - 1, :]], axis=1)
    w = jnp.broadcast_to(eb[:, :, None, :], (hb, _NC, _CH, width))
    w = w.reshape(hb * _NC, _CH, width)
    w = pltpu.roll(w, 1, 2, stride=1, stride_axis=1)
    w = jnp.where(cnp <= rnp, 0.0, w)
    b3 = g3 * w + b3
    return b3.reshape(hb, _M, width)


def _make_attn_kernel(w, rb0, hb):
    def attn(q_ref, k_ref, v_ref, wo_ref, ci_ref, o_ref, co_ref):
        a = [jax.lax.dot_general(q_ref[i], k_ref[i], (((1,), (1,)), ((), ())),
                                 preferred_element_type=jnp.float32)
             for i in range(hb)]
        a = jnp.concatenate([x[None] for x in a], axis=0)

        carry = ci_ref[:, 0:1, :w]
        carry = jnp.concatenate(
            [carry[:, :, _PRE - 1:], jnp.zeros((hb, 1, _PRE - 1), jnp.float32)],
            axis=2)

        g = jnp.concatenate(
            [jnp.zeros((hb, 1, w), jnp.float32),
             jnp.ones((hb, _PRE - 1, w), jnp.float32), a], axis=1)
        b = jnp.concatenate(
            [carry, jnp.zeros((hb, _PRE - 1, w), jnp.float32), a], axis=1)

        b = _scan_stripe(g, b, hb, w)

        last = b[:, _M - 1:_M, :]
        if w == _T:
            co_ref[...] = jnp.broadcast_to(last, (hb, _CH, _T))
        else:
            co_ref[...] = jnp.concatenate(
                [jnp.broadcast_to(last, (hb, _CH, w)),
                 jnp.zeros((hb, _CH, _T - w), jnp.float32)], axis=2)
        y = b[:, _PRE:, :]

        rows = rb0 * _R + jax.lax.broadcasted_iota(jnp.int32, (1, _R, w), 1)
        cols = jax.lax.broadcasted_iota(jnp.int32, (1, _R, w), 2)
        colsf = cols.astype(jnp.float32)
        rinv = (1.0 / _TAU) / (
            rb0 * _R + 1.0
            + jax.lax.broadcasted_iota(jnp.int32, (1, _R, 1), 1)
            .astype(jnp.float32))
        z = jnp.where(cols <= rows, y * (1.0 / _TAU) + colsf * rinv, -jnp.inf)
        z = z - jnp.max(z, axis=2, keepdims=True)
        e = jnp.exp(z)
        probs = e / jnp.sum(e, axis=2, keepdims=True)

        res = [jnp.dot(jnp.dot(probs[i], v_ref[i],
                               preferred_element_type=jnp.float32),
                       wo_ref[i], preferred_element_type=jnp.float32)
               for i in range(hb)]
        o_ref[...] = jnp.concatenate(res, axis=1)

    return attn


def kernel(x, wq, wk, wv, wo):
    x2 = x.reshape(_T, _DIMS)

    wcat = jnp.concatenate([wq, wk, wv], axis=0)
    q_sm, k_sm, v_sm = pl.pallas_call(
        _proj_kernel,
        grid=(_T // _RP,),
        in_specs=[
            pl.BlockSpec((_RP, _DIMS), lambda rb: (rb, 0)),
            pl.BlockSpec((3 * _DIMS, _DIMS), lambda rb: (0, 0)),
        ],
        out_specs=[
            pl.BlockSpec((_NHEADS, _RP, _HD), lambda rb: (0, rb, 0)),
            pl.BlockSpec((_NHEADS, _RP, _HD), lambda rb: (0, rb, 0)),
            pl.BlockSpec((_NHEADS, _RP, _HD), lambda rb: (0, rb, 0)),
        ],
        out_shape=[jax.ShapeDtypeStruct((_NHEADS, _T, _HD), jnp.float32)] * 3,
    )(x2, wcat)

    wo3 = wo.reshape(_NHEADS, _HD, _HD)
    ci = jnp.zeros((_NHEADS, _CH, _T), jnp.float32)
    parts = []
    for rb0 in range(_T // _R):
        w = _R * (rb0 + 1)
        hb = 6 if w <= 256 else (4 if w <= 1024 else 2)
        out_p, co = pl.pallas_call(
            _make_attn_kernel(w, rb0, hb),
            grid=(_NHEADS // hb,),
            in_specs=[
                pl.BlockSpec((hb, _R, _HD), lambda i, rb0=rb0: (i, rb0, 0)),
                pl.BlockSpec((hb, w, _HD), lambda i: (i, 0, 0)),
                pl.BlockSpec((hb, w, _HD), lambda i: (i, 0, 0)),
                pl.BlockSpec((hb, _HD, _HD), lambda i: (i, 0, 0)),
                pl.BlockSpec((hb, _CH, _T), lambda i: (i, 0, 0)),
            ],
            out_specs=[
                pl.BlockSpec((_R, hb * _HD), lambda i: (0, i)),
                pl.BlockSpec((hb, _CH, _T), lambda i: (i, 0, 0)),
            ],
            out_shape=[
                jax.ShapeDtypeStruct((_R, _DIMS), jnp.float32),
                jax.ShapeDtypeStruct((_NHEADS, _CH, _T), jnp.float32),
            ],
        )(q_sm, k_sm, v_sm, wo3, ci)
        parts.append(out_p)
        ci = co

    return jnp.concatenate(parts, axis=0).reshape(_B, _T, _DIMS)
